# Initial kernel scaffold; baseline (speedup 1.0000x reference)
import jax
import jax.numpy as jnp
from jax.experimental import pallas as pl


def kernel(x, edge_index, edge_attr, params):
    raise NotImplementedError("write your pallas kernel here")


# speedup vs baseline: 1.4722x; 1.4722x over previous
import functools

import jax
import jax.numpy as jnp
from jax import lax
from jax.experimental import pallas as pl
from jax.experimental.pallas import tpu as pltpu
from jax.experimental.pallas import tpu_sc as plsc

N = 50000
E = 800000
H = 64
NTC = 50176
EP = 802816
NACC = 51200
TEDGE = 1024
TNODE = 1024
GE = EP // TEDGE
GN = NTC // TNODE
NC, NS = 2, 16
EPT = EP // NS
CHUNKS = EPT // 128
ROWS_PT = NACC // NS
IDXC = 56

_SQRT1_2 = 0.7071067811865476


def _gelu(t):
    return t * 0.5 * (1.0 + lax.erf(t * _SQRT1_2))


def _lnk(t, g, b):
    m = jnp.mean(t, axis=-1, keepdims=True)
    v = jnp.mean((t - m) ** 2, axis=-1, keepdims=True)
    return (t - m) * lax.rsqrt(v + 1e-5) * g + b


def _dot(a, b):
    return lax.dot_general(a, b, (((1,), (0,)), ((), ())),
                           preferred_element_type=jnp.float32)


def _wspec(shape):
    nd = len(shape)
    return pl.BlockSpec(shape, lambda i: (0,) * nd)


def _gather_body(x_hbm, idx_hbm, g_hbm, idx_v, rows_v, sem):
    c = lax.axis_index("c")
    s = lax.axis_index("s")
    pltpu.sync_copy(idx_hbm.at[c, s], idx_v)
    base = s * EPT

    def body(j, carry):
        pltpu.async_copy(x_hbm.at[idx_v.at[j]], rows_v, sem).wait()
        pltpu.sync_copy(rows_v, g_hbm.at[c, pl.ds(base + j * 128, 128)])
        return carry

    lax.fori_loop(0, CHUNKS, body, 0)


def _scatter_body(e_hbm, dst_hbm, z_hbm, out_hbm, idx_v, ebuf, acc):
    c = lax.axis_index("c")
    s = lax.axis_index("s")
    pltpu.sync_copy(z_hbm, acc.at[pl.ds(s * ROWS_PT, ROWS_PT)])
    plsc.subcore_barrier()
    base = s * EPT

    def outer(k, carry):
        pltpu.sync_copy(dst_hbm.at[s, pl.ds(k * IDXC, IDXC)], idx_v)

        def body(j, carry2):
            pltpu.sync_copy(
                e_hbm.at[c, pl.ds(base + (k * IDXC + j) * 128, 128)], ebuf)
            pltpu.sync_copy(ebuf, acc.at[idx_v.at[j]], add=True)
            return carry2

        lax.fori_loop(0, IDXC, body, carry)
        return carry

    lax.fori_loop(0, CHUNKS // IDXC, outer, 0)
    plsc.subcore_barrier()
    pltpu.sync_copy(acc.at[pl.ds(s * ROWS_PT, ROWS_PT)],
                    out_hbm.at[c, pl.ds(s * ROWS_PT, ROWS_PT)])


@functools.cache
def _sc_kernels():
    mesh = plsc.VectorSubcoreMesh(core_axis_name="c", subcore_axis_name="s",
                                  num_cores=NC, num_subcores=NS)
    gather = pl.kernel(
        _gather_body,
        out_type=jax.ShapeDtypeStruct((2, EP, H), jnp.float32),
        mesh=mesh,
        scratch_types=[
            pltpu.VMEM((CHUNKS, 128), jnp.int32),
            pltpu.VMEM((128, H), jnp.float32),
            pltpu.SemaphoreType.DMA,
        ],
        compiler_params=pltpu.CompilerParams(use_tc_tiling_on_sc=False),
    )
    scatter = pl.kernel(
        _scatter_body,
        out_type=jax.ShapeDtypeStruct((2, NACC, 32), jnp.float32),
        mesh=mesh,
        scratch_types=[
            pltpu.VMEM((IDXC, 128), jnp.int32),
            pltpu.VMEM((128, 32), jnp.float32),
            pltpu.VMEM_SHARED((NACC, 32), jnp.float32),
        ],
        compiler_params=pltpu.CompilerParams(use_tc_tiling_on_sc=False),
    )
    return gather, scatter


def _gather_sc(xc, idxg):
    return _sc_kernels()[0](xc, idxg)


def _scatter_sc(e2, dst_s, zrows):
    return _sc_kernels()[1](e2, dst_s, zrows)


def _input_mlp(x8, w1, b1, g1, e1, w2, b2, g2, e2):
    def body(x_ref, w1r, b1r, g1r, e1r, w2r, b2r, g2r, e2r, o_ref):
        h = _lnk(_dot(x_ref[...], w1r[...]) + b1r[...], g1r[...], e1r[...])
        h = _gelu(h)
        o_ref[...] = _lnk(_dot(h, w2r[...]) + b2r[...], g2r[...], e2r[...])

    return pl.pallas_call(
        body,
        grid=(GN,),
        in_specs=[pl.BlockSpec((TNODE, 8), lambda i: (i, 0)),
                  _wspec((8, H)), _wspec((1, H)), _wspec((1, H)), _wspec((1, H)),
                  _wspec((H, H)), _wspec((1, H)), _wspec((1, H)), _wspec((1, H))],
        out_specs=pl.BlockSpec((TNODE, H), lambda i: (i, 0)),
        out_shape=jax.ShapeDtypeStruct((NTC, H), jnp.float32),
    )(x8, w1, b1, g1, e1, w2, b2, g2, e2)


def _edge_mlp(g, ea8, wa, wb, wc, b1, g1, e1, w2, b2, g2, e2):
    def body(g_ref, ea_ref, war, wbr, wcr, b1r, g1r, e1r, w2r, b2r, g2r, e2r,
             o_ref):
        m = (_dot(g_ref[0], war[...]) + _dot(g_ref[1], wbr[...])
             + _dot(ea_ref[...], wcr[...]) + b1r[...])
        e = m[:, :H] * m[:, H:]
        e = _gelu(_lnk(e, g1r[...], e1r[...]))
        e = _dot(e, w2r[...]) + b2r[...]
        e = _gelu(_lnk(e, g2r[...], e2r[...]))
        o_ref[0] = e[:, :32]
        o_ref[1] = e[:, 32:]

    return pl.pallas_call(
        body,
        grid=(GE,),
        in_specs=[pl.BlockSpec((2, TEDGE, H), lambda i: (0, i, 0)),
                  pl.BlockSpec((TEDGE, 8), lambda i: (i, 0)),
                  _wspec((H, 2 * H)), _wspec((H, 2 * H)), _wspec((8, 2 * H)),
                  _wspec((1, 2 * H)), _wspec((1, H)), _wspec((1, H)),
                  _wspec((H, H)), _wspec((1, H)), _wspec((1, H)), _wspec((1, H))],
        out_specs=pl.BlockSpec((2, TEDGE, 32), lambda i: (0, i, 0)),
        out_shape=jax.ShapeDtypeStruct((2, EP, 32), jnp.float32),
    )(g, ea8, wa, wb, wc, b1, g1, e1, w2, b2, g2, e2)


def _node_mlp(x, agg, wx, wa0, wa1, b1, g1, e1, w2, b2, g2, e2):
    def body(x_ref, a_ref, wxr, wa0r, wa1r, b1r, g1r, e1r, w2r, b2r, g2r, e2r,
             o_ref):
        xv = x_ref[...]
        u = (_dot(xv, wxr[...]) + _dot(a_ref[0], wa0r[...])
             + _dot(a_ref[1], wa1r[...]) + b1r[...])
        u = _gelu(_lnk(u, g1r[...], e1r[...]))
        u = _lnk(_dot(u, w2r[...]) + b2r[...], g2r[...], e2r[...])
        o_ref[...] = _gelu(xv + u)

    return pl.pallas_call(
        body,
        grid=(GN,),
        in_specs=[pl.BlockSpec((TNODE, H), lambda i: (i, 0)),
                  pl.BlockSpec((2, TNODE, 32), lambda i: (0, i, 0)),
                  _wspec((H, H)), _wspec((32, H)), _wspec((32, H)),
                  _wspec((1, H)), _wspec((1, H)), _wspec((1, H)),
                  _wspec((H, H)), _wspec((1, H)), _wspec((1, H)), _wspec((1, H))],
        out_specs=pl.BlockSpec((TNODE, H), lambda i: (i, 0)),
        out_shape=jax.ShapeDtypeStruct((NTC, H), jnp.float32),
    )(x, agg, wx, wa0, wa1, b1, g1, e1, w2, b2, g2, e2)


def _out_mlp(x, w1, b1, w2, b2):
    def body(x_ref, w1r, b1r, w2r, b2r, o_ref):
        h = _gelu(_dot(x_ref[...], w1r[...]) + b1r[...])
        o_ref[...] = _dot(h, w2r[...]) + b2r[...]

    return pl.pallas_call(
        body,
        grid=(GN,),
        in_specs=[pl.BlockSpec((TNODE, H), lambda i: (i, 0)),
                  _wspec((H, H)), _wspec((1, H)), _wspec((H, 8)), _wspec((1, 8))],
        out_specs=pl.BlockSpec((TNODE, 8), lambda i: (i, 0)),
        out_shape=jax.ShapeDtypeStruct((NTC, 8), jnp.float32),
    )(x, w1, b1, w2, b2)


def kernel(x, edge_index, edge_attr, params):
    p = params
    f32 = jnp.float32

    x8 = jnp.zeros((NTC, 8), f32).at[:N, :5].set(x)
    src = edge_index[0]
    dst = edge_index[1]
    padg = EP - E
    zpad = jnp.zeros((padg,), jnp.int32)
    idxg = jnp.stack([jnp.concatenate([dst, zpad]),
                      jnp.concatenate([src, zpad])]).reshape(2, NS, CHUNKS, 128)
    dst_s = jnp.concatenate(
        [dst, jnp.full((padg,), N, jnp.int32)]).reshape(NS, CHUNKS, 128)
    ea8 = jnp.zeros((EP, 8), f32).at[:E, :4].set(edge_attr)
    zrows = jnp.zeros((ROWS_PT, 32), f32)

    iw1 = jnp.zeros((8, H), f32).at[:5].set(p['iW1'])
    xc = _input_mlp(x8, iw1, p['ib1'][None], p['iln1_g'][None],
                    p['iln1_b'][None], p['iW2'], p['ib2'][None],
                    p['iln2_g'][None], p['iln2_b'][None])

    for l in range(3):
        g = _gather_sc(xc, idxg)
        wc = jnp.zeros((8, 2 * H), f32).at[:4].set(p['eW1'][l][2 * H:])
        e2 = _edge_mlp(g, ea8,
                       p['eW1'][l][:H], p['eW1'][l][H:2 * H], wc,
                       p['eb1'][l][None], p['eln1_g'][l][None],
                       p['eln1_b'][l][None], p['eW2'][l], p['eb2'][l][None],
                       p['eln2_g'][l][None], p['eln2_b'][l][None])
        agg = _scatter_sc(e2, dst_s, zrows)
        xc = _node_mlp(xc, agg,
                       p['nW1'][l][:H], p['nW1'][l][H:H + 32],
                       p['nW1'][l][H + 32:], p['nb1'][l][None],
                       p['nln1_g'][l][None], p['nln1_b'][l][None],
                       p['nW2'][l], p['nb2'][l][None],
                       p['nln2_g'][l][None], p['nln2_b'][l][None])

    ow2 = jnp.zeros((H, 8), f32).at[:, :1].set(p['oW2'])
    ob2 = jnp.zeros((1, 8), f32).at[0, 0].set(p['ob2'][0])
    out8 = _out_mlp(xc, p['oW1'], p['ob1'][None], ow2, ob2)
    return out8[:N, 0]


# speedup vs baseline: 1.9247x; 1.3074x over previous
import functools

import jax
import jax.numpy as jnp
from jax import lax
from jax.experimental import pallas as pl
from jax.experimental.pallas import tpu as pltpu
from jax.experimental.pallas import tpu_sc as plsc

N = 50000
E = 800000
H = 64
NTC = 50176
EP = 802816
NACC = 51200
TEDGE = 1024
TNODE = 1024
GE = EP // TEDGE
GN = NTC // TNODE
NC, NS = 2, 16
EPT = EP // NS
CHUNKS = EPT // 128
EPW = EP // (NC * NS)
WCHUNKS = EPW // 128
ROWS_PT = NACC // NS
IDXC = 56

_SQRT1_2 = 0.7071067811865476


def _gelu(t):
    return t * 0.5 * (1.0 + lax.erf(t * _SQRT1_2))


def _lnk(t, g, b):
    m = jnp.mean(t, axis=-1, keepdims=True)
    v = jnp.mean((t - m) ** 2, axis=-1, keepdims=True)
    return (t - m) * lax.rsqrt(v + 1e-5) * g + b


def _dot(a, b):
    return lax.dot_general(a, b, (((1,), (0,)), ((), ())),
                           preferred_element_type=jnp.float32)


def _wspec(shape):
    nd = len(shape)
    return pl.BlockSpec(shape, lambda i: (0,) * nd)


def _gather_body(xa_hbm, xb_hbm, ia_hbm, ib_hbm, m_hbm, ia_v, ib_v, rows_v,
                 sem):
    c = lax.axis_index("c")
    s = lax.axis_index("s")
    w = s * NC + c
    pltpu.sync_copy(ia_hbm.at[w], ia_v)
    pltpu.sync_copy(ib_hbm.at[w], ib_v)
    base = w * EPW

    def body(j, carry):
        pltpu.async_copy(xa_hbm.at[ia_v.at[j]], rows_v, sem).wait()
        pltpu.async_copy(xb_hbm.at[ib_v.at[j]], rows_v, sem, add=True).wait()
        pltpu.sync_copy(rows_v, m_hbm.at[pl.ds(base + j * 128, 128)])
        return carry

    lax.fori_loop(0, WCHUNKS, body, 0)


def _scatter_body(e_hbm, dst_hbm, z_hbm, out_hbm, idx_v, ebuf, acc):
    c = lax.axis_index("c")
    s = lax.axis_index("s")
    pltpu.sync_copy(z_hbm, acc.at[pl.ds(s * ROWS_PT, ROWS_PT)])
    plsc.subcore_barrier()
    base = s * EPT

    def outer(k, carry):
        pltpu.sync_copy(dst_hbm.at[s, pl.ds(k * IDXC, IDXC)], idx_v)

        def body(j, carry2):
            pltpu.sync_copy(
                e_hbm.at[c, pl.ds(base + (k * IDXC + j) * 128, 128)], ebuf)
            pltpu.sync_copy(ebuf, acc.at[idx_v.at[j]], add=True)
            return carry2

        lax.fori_loop(0, IDXC, body, carry)
        return carry

    lax.fori_loop(0, CHUNKS // IDXC, outer, 0)
    plsc.subcore_barrier()
    pltpu.sync_copy(acc.at[pl.ds(s * ROWS_PT, ROWS_PT)],
                    out_hbm.at[c, pl.ds(s * ROWS_PT, ROWS_PT)])


@functools.cache
def _sc_kernels():
    mesh = plsc.VectorSubcoreMesh(core_axis_name="c", subcore_axis_name="s",
                                  num_cores=NC, num_subcores=NS)
    gather = pl.kernel(
        _gather_body,
        out_type=jax.ShapeDtypeStruct((EP, 2 * H), jnp.float32),
        mesh=mesh,
        scratch_types=[
            pltpu.VMEM((WCHUNKS, 128), jnp.int32),
            pltpu.VMEM((WCHUNKS, 128), jnp.int32),
            pltpu.VMEM((128, 2 * H), jnp.float32),
            pltpu.SemaphoreType.DMA,
        ],
    )
    scatter = pl.kernel(
        _scatter_body,
        out_type=jax.ShapeDtypeStruct((2, NACC, 32), jnp.float32),
        mesh=mesh,
        scratch_types=[
            pltpu.VMEM((IDXC, 128), jnp.int32),
            pltpu.VMEM((128, 32), jnp.float32),
            pltpu.VMEM_SHARED((NACC, 32), jnp.float32),
        ],
        compiler_params=pltpu.CompilerParams(use_tc_tiling_on_sc=False),
    )
    return gather, scatter


def _gather_sc(xa, xb, ia, ib):
    return _sc_kernels()[0](xa, xb, ia, ib)


def _scatter_sc(e2, dst_s, zrows):
    return _sc_kernels()[1](e2, dst_s, zrows)


def _input_mlp(x8, w1, b1, g1, e1, w2, b2, g2, e2):
    def body(x_ref, w1r, b1r, g1r, e1r, w2r, b2r, g2r, e2r, o_ref):
        h = _lnk(_dot(x_ref[...], w1r[...]) + b1r[...], g1r[...], e1r[...])
        h = _gelu(h)
        o_ref[...] = _lnk(_dot(h, w2r[...]) + b2r[...], g2r[...], e2r[...])

    return pl.pallas_call(
        body,
        grid=(GN,),
        in_specs=[pl.BlockSpec((TNODE, 8), lambda i: (i, 0)),
                  _wspec((8, H)), _wspec((1, H)), _wspec((1, H)), _wspec((1, H)),
                  _wspec((H, H)), _wspec((1, H)), _wspec((1, H)), _wspec((1, H))],
        out_specs=pl.BlockSpec((TNODE, H), lambda i: (i, 0)),
        out_shape=jax.ShapeDtypeStruct((NTC, H), jnp.float32),
    )(x8, w1, b1, g1, e1, w2, b2, g2, e2)


def _pre_mlp(x, wa, wb, b1):
    def body(x_ref, war, wbr, b1r, oa_ref, ob_ref):
        xv = x_ref[...]
        oa_ref[...] = _dot(xv, war[...]) + b1r[...]
        ob_ref[...] = _dot(xv, wbr[...])

    return pl.pallas_call(
        body,
        grid=(GN,),
        in_specs=[pl.BlockSpec((TNODE, H), lambda i: (i, 0)),
                  _wspec((H, 2 * H)), _wspec((H, 2 * H)), _wspec((1, 2 * H))],
        out_specs=[pl.BlockSpec((TNODE, 2 * H), lambda i: (i, 0)),
                   pl.BlockSpec((TNODE, 2 * H), lambda i: (i, 0))],
        out_shape=[jax.ShapeDtypeStruct((NTC, 2 * H), jnp.float32),
                   jax.ShapeDtypeStruct((NTC, 2 * H), jnp.float32)],
    )(x, wa, wb, b1)


def _edge_mlp(m, eap, wc, g1, e1, w2, b2, g2, e2):
    def body(m_ref, ea_ref, wcr, g1r, e1r, w2r, b2r, g2r, e2r, o_ref):
        eab = ea_ref[...]
        ea = jnp.concatenate([eab[:, 8 * k:8 * (k + 1)] for k in range(16)],
                             axis=0)
        mm = m_ref[...] + _dot(ea, wcr[...])
        e = mm[:, :H] * mm[:, H:]
        e = _gelu(_lnk(e, g1r[...], e1r[...]))
        e = _dot(e, w2r[...]) + b2r[...]
        e = _gelu(_lnk(e, g2r[...], e2r[...]))
        o_ref[0] = e[:, :32]
        o_ref[1] = e[:, 32:]

    return pl.pallas_call(
        body,
        grid=(GE,),
        in_specs=[pl.BlockSpec((TEDGE, 2 * H), lambda i: (i, 0)),
                  pl.BlockSpec((TEDGE // 16, 128), lambda i: (i, 0)),
                  _wspec((8, 2 * H)), _wspec((1, H)), _wspec((1, H)),
                  _wspec((H, H)), _wspec((1, H)), _wspec((1, H)), _wspec((1, H))],
        out_specs=pl.BlockSpec((2, TEDGE, 32), lambda i: (0, i, 0)),
        out_shape=jax.ShapeDtypeStruct((2, EP, 32), jnp.float32),
    )(m, eap, wc, g1, e1, w2, b2, g2, e2)


def _node_mlp(x, agg, wx, wa0, wa1, b1, g1, e1, w2, b2, g2, e2):
    def body(x_ref, a_ref, wxr, wa0r, wa1r, b1r, g1r, e1r, w2r, b2r, g2r, e2r,
             o_ref):
        xv = x_ref[...]
        u = (_dot(xv, wxr[...]) + _dot(a_ref[0], wa0r[...])
             + _dot(a_ref[1], wa1r[...]) + b1r[...])
        u = _gelu(_lnk(u, g1r[...], e1r[...]))
        u = _lnk(_dot(u, w2r[...]) + b2r[...], g2r[...], e2r[...])
        o_ref[...] = _gelu(xv + u)

    return pl.pallas_call(
        body,
        grid=(GN,),
        in_specs=[pl.BlockSpec((TNODE, H), lambda i: (i, 0)),
                  pl.BlockSpec((2, TNODE, 32), lambda i: (0, i, 0)),
                  _wspec((H, H)), _wspec((32, H)), _wspec((32, H)),
                  _wspec((1, H)), _wspec((1, H)), _wspec((1, H)),
                  _wspec((H, H)), _wspec((1, H)), _wspec((1, H)), _wspec((1, H))],
        out_specs=pl.BlockSpec((TNODE, H), lambda i: (i, 0)),
        out_shape=jax.ShapeDtypeStruct((NTC, H), jnp.float32),
    )(x, agg, wx, wa0, wa1, b1, g1, e1, w2, b2, g2, e2)


def _out_mlp(x, w1, b1, w2, b2):
    def body(x_ref, w1r, b1r, w2r, b2r, o_ref):
        h = _gelu(_dot(x_ref[...], w1r[...]) + b1r[...])
        o_ref[...] = _dot(h, w2r[...]) + b2r[...]

    return pl.pallas_call(
        body,
        grid=(GN,),
        in_specs=[pl.BlockSpec((TNODE, H), lambda i: (i, 0)),
                  _wspec((H, H)), _wspec((1, H)), _wspec((H, 8)), _wspec((1, 8))],
        out_specs=pl.BlockSpec((TNODE, 8), lambda i: (i, 0)),
        out_shape=jax.ShapeDtypeStruct((NTC, 8), jnp.float32),
    )(x, w1, b1, w2, b2)


def kernel(x, edge_index, edge_attr, params):
    p = params
    f32 = jnp.float32

    x8 = jnp.zeros((NTC, 8), f32).at[:N, :5].set(x)
    src = edge_index[0]
    dst = edge_index[1]
    padg = EP - E
    zpad = jnp.zeros((padg,), jnp.int32)
    ia = jnp.concatenate([dst, zpad]).reshape(NC * NS, WCHUNKS, 128)
    ib = jnp.concatenate([src, zpad]).reshape(NC * NS, WCHUNKS, 128)
    dst_s = jnp.concatenate(
        [dst, jnp.full((padg,), N, jnp.int32)]).reshape(NS, CHUNKS, 128)
    ea8 = jnp.zeros((EP, 8), f32).at[:E, :4].set(edge_attr)
    eap = ea8.reshape(GE, 16, 64, 8).transpose(0, 2, 1, 3).reshape(EP // 16, 128)
    zrows = jnp.zeros((ROWS_PT, 32), f32)

    iw1 = jnp.zeros((8, H), f32).at[:5].set(p['iW1'])
    xc = _input_mlp(x8, iw1, p['ib1'][None], p['iln1_g'][None],
                    p['iln1_b'][None], p['iW2'], p['ib2'][None],
                    p['iln2_g'][None], p['iln2_b'][None])

    for l in range(3):
        xa, xb = _pre_mlp(xc, p['eW1'][l][:H], p['eW1'][l][H:2 * H],
                          p['eb1'][l][None])
        m = _gather_sc(xa, xb, ia, ib)
        wc = jnp.zeros((8, 2 * H), f32).at[:4].set(p['eW1'][l][2 * H:])
        e2 = _edge_mlp(m, eap, wc,
                       p['eln1_g'][l][None], p['eln1_b'][l][None],
                       p['eW2'][l], p['eb2'][l][None],
                       p['eln2_g'][l][None], p['eln2_b'][l][None])
        agg = _scatter_sc(e2, dst_s, zrows)
        xc = _node_mlp(xc, agg,
                       p['nW1'][l][:H], p['nW1'][l][H:H + 32],
                       p['nW1'][l][H + 32:], p['nb1'][l][None],
                       p['nln1_g'][l][None], p['nln1_b'][l][None],
                       p['nW2'][l], p['nb2'][l][None],
                       p['nln2_g'][l][None], p['nln2_b'][l][None])

    ow2 = jnp.zeros((H, 8), f32).at[:, :1].set(p['oW2'])
    ob2 = jnp.zeros((1, 8), f32).at[0, 0].set(p['ob2'][0])
    out8 = _out_mlp(xc, p['oW1'], p['ob1'][None], ow2, ob2)
    return out8[:N, 0]
